# speedup vs baseline: 10.6929x; 10.6929x over previous
import functools
import jax
import jax.numpy as jnp
from jax.experimental import pallas as pl
from jax.experimental.pallas import tpu as pltpu

E = 64
K = 2
D = 768
H = 256
O = 768
T = 512
A = T * K
CHUNK = 128


def _gather_body(tok_sm, hid_ref, xs_ref):
    def row(j, c):
        xs_ref[j, :] = hid_ref[tok_sm[j], :]
        return c
    jax.lax.fori_loop(0, A, row, 0)


def _moe_body(offs_sm, xs_ref, gu_ref, dw_ref, w_ref, ys_ref):
    e = pl.program_id(0)

    @pl.when(e == 0)
    def _():
        ys_ref[...] = jnp.zeros_like(ys_ref)

    start = offs_sm[e]
    end = offs_sm[e + 1]
    a0 = (start // CHUNK) * CHUNK
    nchunks = jnp.where(end > start, (end - a0 + CHUNK - 1) // CHUNK, 0)

    def chunk(c, carry):
        cs = pl.multiple_of(a0 + c * CHUNK, CHUNK)
        x = xs_ref[pl.ds(cs, CHUNK), :]
        gu = jax.lax.dot_general(
            x, gu_ref[0], (((1,), (1,)), ((), ())),
            preferred_element_type=jnp.float32)
        g = gu[:, :H]
        u = gu[:, H:]
        h = g * jax.nn.sigmoid(g) * u
        y = jax.lax.dot_general(
            h, dw_ref[0], (((1,), (1,)), ((), ())),
            preferred_element_type=jnp.float32)
        rows = cs + jax.lax.broadcasted_iota(jnp.int32, (CHUNK, 1), 0)
        scale = jnp.where((rows >= start) & (rows < end),
                          w_ref[pl.ds(cs, CHUNK), :], 0.0)
        ys_ref[pl.ds(cs, CHUNK), :] += y * scale
        return carry

    jax.lax.fori_loop(0, nchunks, chunk, 0)


def _combine_body(inv_sm, ys_ref, out_ref):
    def row(t, c):
        ia = inv_sm[2 * t]
        ib = inv_sm[2 * t + 1]
        out_ref[t, :] = ys_ref[ia, :] + ys_ref[ib, :]
        return c
    jax.lax.fori_loop(0, T, row, 0)


@jax.jit
def kernel(hidden_states, top_k_indices, top_k_weights, gate_up_proj, down_proj):
    flat_idx = top_k_indices.reshape(-1).astype(jnp.int32)
    order = jnp.argsort(flat_idx).astype(jnp.int32)
    sorted_idx = flat_idx[order]
    sorted_tok = (order // K).astype(jnp.int32)
    sorted_w = top_k_weights.reshape(-1)[order].reshape(A, 1)
    offsets = jnp.searchsorted(sorted_idx, jnp.arange(E + 1, dtype=jnp.int32)
                               ).astype(jnp.int32)
    inv = jnp.argsort(order).astype(jnp.int32)

    x_sorted = pl.pallas_call(
        _gather_body,
        grid_spec=pltpu.PrefetchScalarGridSpec(
            num_scalar_prefetch=1,
            grid=(1,),
            in_specs=[pl.BlockSpec((T, D), lambda i, s: (0, 0))],
            out_specs=pl.BlockSpec((A, D), lambda i, s: (0, 0)),
        ),
        out_shape=jax.ShapeDtypeStruct((A, D), jnp.float32),
    )(sorted_tok, hidden_states)

    y_sorted = pl.pallas_call(
        _moe_body,
        grid_spec=pltpu.PrefetchScalarGridSpec(
            num_scalar_prefetch=1,
            grid=(E,),
            in_specs=[
                pl.BlockSpec((A, D), lambda e, s: (0, 0)),
                pl.BlockSpec((1, 2 * H, D), lambda e, s: (e, 0, 0)),
                pl.BlockSpec((1, O, H), lambda e, s: (e, 0, 0)),
                pl.BlockSpec((A, 1), lambda e, s: (0, 0)),
            ],
            out_specs=pl.BlockSpec((A, O), lambda e, s: (0, 0)),
        ),
        out_shape=jax.ShapeDtypeStruct((A, O), jnp.float32),
    )(offsets, x_sorted, gate_up_proj, down_proj, sorted_w)

    output = pl.pallas_call(
        _combine_body,
        grid_spec=pltpu.PrefetchScalarGridSpec(
            num_scalar_prefetch=1,
            grid=(1,),
            in_specs=[pl.BlockSpec((A, O), lambda i, s: (0, 0))],
            out_specs=pl.BlockSpec((T, O), lambda i, s: (0, 0)),
        ),
        out_shape=jax.ShapeDtypeStruct((T, O), jnp.float32),
    )(inv, y_sorted)

    return output
